# baseline (speedup 1.0000x reference)
import functools

import jax
import jax.numpy as jnp
from jax import lax
from jax.experimental import pallas as pl
from jax.experimental.pallas import tpu as pltpu
from jax.experimental.pallas import tpu_sc as plsc

N = 4096
D = 1024
P = 128
C = 2 * P
NW = 32
CPW = P // NW
NV = N // 16

MININT = -2147483648


def _projections():
    k = jax.random.key(0)
    proj = jax.random.normal(k, (D, P), dtype=jnp.float32)
    proj = proj / jnp.sqrt(jnp.sum(proj**2, axis=0, keepdims=True))
    return proj


def _mm_body(x_ref, y_ref, p_ref, bt_ref):
    dn = (((0,), (1,)), ((), ()))
    bt_ref[:P, :] = lax.dot_general(p_ref[...], x_ref[...], dn,
                                    preferred_element_type=jnp.float32)
    bt_ref[P:, :] = lax.dot_general(p_ref[...], y_ref[...], dn,
                                    preferred_element_type=jnp.float32)


def _project(X, Y, proj):
    vmem = pl.BlockSpec(memory_space=pltpu.VMEM)
    return pl.pallas_call(
        _mm_body,
        in_specs=[vmem, vmem, vmem],
        out_specs=pl.BlockSpec(memory_space=pltpu.VMEM),
        out_shape=jax.ShapeDtypeStruct((C, N), jnp.float32),
    )(X, Y, proj)


def _full(v):
    return jnp.full((16,), v, jnp.int32)


def _transform(fbuf, dst):
    def body(i, carry):
        sl = pl.ds(i * 16, 16)
        u = lax.bitcast_convert_type(fbuf[sl], jnp.int32)
        s = lax.shift_right_arithmetic(u, _full(31))
        dst[sl] = lax.bitwise_xor(u, lax.bitwise_or(s, _full(MININT)))
        return carry
    lax.fori_loop(0, NV, body, 0, unroll=8)


def _hist_pass(src, hist, sh, base, iota16):
    def body(i, carry):
        k = plsc.load_gather(src, [base + i])
        d = lax.bitwise_and(lax.shift_right_logical(k, _full(sh)), _full(255))
        plsc.addupdate_scatter(hist, [d * 16 + iota16], _full(1))
        return carry
    lax.fori_loop(0, NV, body, 0, unroll=8)


def _offsets(hist, off):
    def body(v, carry):
        sl = pl.ds(v * 16, 16)
        h = hist[sl]
        cs = plsc.cumsum(h)
        off[sl] = cs - h + carry
        hist[sl] = _full(0)
        return carry + jnp.sum(h)
    lax.fori_loop(0, NV, body, jnp.int32(0), unroll=8)


def _permute2(sx, dx, ox, sy, dy, oy, sh, base, iota16):
    def one(src, dst, off, i):
        k = plsc.load_gather(src, [base + i])
        d = lax.bitwise_and(lax.shift_right_logical(k, _full(sh)), _full(255))
        didx = d * 16 + iota16
        pos = lax.bitwise_and(plsc.load_gather(off, [didx]), _full(N - 1))
        plsc.store_scatter(dst, [pos], k)
        plsc.addupdate_scatter(off, [didx], _full(1))

    def body(i, carry):
        one(sx, dx, ox, i)
        one(sy, dy, oy, i)
        return carry
    lax.fori_loop(0, NV, body, 0, unroll=8)


def _sort_pair(fxq, fyq, a, b, c, d_, hx, hy, ox, oy, base, iota16):
    _transform(fxq, a)
    _transform(fyq, c)
    _hist_pass(a, hx, 0, base, iota16)
    _hist_pass(c, hy, 0, base, iota16)
    _offsets(hx, ox)
    _offsets(hy, oy)
    _permute2(a, b, ox, c, d_, oy, 0, base, iota16)
    for sh, sx, dx, sy, dy in ((8, b, a, d_, c), (16, a, b, c, d_),
                               (24, b, a, d_, c)):
        _hist_pass(sx, hx, sh, base, iota16)
        _hist_pass(sy, hy, sh, base, iota16)
        _offsets(hx, ox)
        _offsets(hy, oy)
        _permute2(sx, dx, ox, sy, dy, oy, sh, base, iota16)


def _k2f(k):
    s = lax.shift_right_arithmetic(lax.bitwise_not(k), _full(31))
    return lax.bitcast_convert_type(
        lax.bitwise_xor(k, lax.bitwise_or(s, _full(MININT))), jnp.float32)


def _cost(a, c):
    def body(i, acc):
        sl = pl.ds(i * 16, 16)
        d = _k2f(a[sl]) - _k2f(c[sl])
        return acc + d * d
    return lax.fori_loop(0, NV, body, jnp.zeros((16,), jnp.float32), unroll=8)


def _sc_body(bt_hbm, out_hbm, fx, fy, a, b, c, d_, hx, hy, ox, oy, accv):
    wid = lax.axis_index("s") * 2 + lax.axis_index("c")
    iota16 = lax.iota(jnp.int32, 16)
    base = iota16 * NV

    pltpu.sync_copy(bt_hbm.at[pl.ds(wid * CPW * N, CPW * N)], fx)
    pltpu.sync_copy(bt_hbm.at[pl.ds((P + wid * CPW) * N, CPW * N)], fy)

    accv[...] = jnp.zeros((16,), jnp.float32)

    def clr(i, carry):
        hx[pl.ds(i * 16, 16)] = _full(0)
        hy[pl.ds(i * 16, 16)] = _full(0)
        return carry
    lax.fori_loop(0, NV, clr, 0, unroll=8)

    def per_pair(q, carry):
        _sort_pair(fx.at[pl.ds(q * N, N)], fy.at[pl.ds(q * N, N)],
                   a, b, c, d_, hx, hy, ox, oy, base, iota16)
        accv[...] += _cost(a, c)
        return carry
    lax.fori_loop(0, CPW, per_pair, 0)

    pltpu.sync_copy(accv, out_hbm.at[pl.ds(wid * 16, 16)])


def _sc_sort_cost(BT):
    mesh = plsc.VectorSubcoreMesh(core_axis_name="c", subcore_axis_name="s")
    f = functools.partial(
        pl.kernel,
        mesh=mesh,
        compiler_params=pltpu.CompilerParams(needs_layout_passes=False),
        out_type=jax.ShapeDtypeStruct((NW * 16,), jnp.float32),
        scratch_types=[
            pltpu.VMEM((CPW * N,), jnp.float32),
            pltpu.VMEM((CPW * N,), jnp.float32),
            pltpu.VMEM((N,), jnp.int32),
            pltpu.VMEM((N,), jnp.int32),
            pltpu.VMEM((N,), jnp.int32),
            pltpu.VMEM((N,), jnp.int32),
            pltpu.VMEM((N,), jnp.int32),
            pltpu.VMEM((N,), jnp.int32),
            pltpu.VMEM((N,), jnp.int32),
            pltpu.VMEM((N,), jnp.int32),
            pltpu.VMEM((16,), jnp.float32),
        ],
    )(_sc_body)
    return f(BT.reshape(C * N))


def _fin_body(c_ref, o_ref):
    s = jnp.sum(c_ref[...], keepdims=True).reshape(1, 1)
    o_ref[...] = jnp.sqrt(s * (1.0 / (N * P)))


def _finalize(costs):
    return pl.pallas_call(
        _fin_body,
        in_specs=[pl.BlockSpec(memory_space=pltpu.VMEM)],
        out_specs=pl.BlockSpec(memory_space=pltpu.VMEM),
        out_shape=jax.ShapeDtypeStruct((1, 1), jnp.float32),
    )(costs)


def kernel(X, Y):
    proj = _projections()
    BT = _project(X, Y, proj)
    costs = _sc_sort_cost(BT)
    return _finalize(costs.reshape(8, 64))[0, 0]


# speedup vs baseline: 2.4107x; 2.4107x over previous
import functools

import jax
import jax.numpy as jnp
from jax import lax
from jax.experimental import pallas as pl
from jax.experimental.pallas import tpu as pltpu
from jax.experimental.pallas import tpu_sc as plsc

N = 4096
D = 1024
P = 128
SCP = 32
TCP = P - SCP
CT = 2 * TCP
NW = 32
NV = N // 16

MININT = -2147483648

_SCHED = [(k, j) for k in range(1, 13) for j in range(k - 1, -1, -1)]
N_SUB = len(_SCHED)


def _projections():
    k = jax.random.key(0)
    proj = jax.random.normal(k, (D, P), dtype=jnp.float32)
    proj = proj / jnp.sqrt(jnp.sum(proj**2, axis=0, keepdims=True))
    return proj


def _mm_body(x_ref, y_ref, psc_ref, ptc_ref, btsc_ref, btc_ref):
    dn = (((0,), (1,)), ((), ()))
    btsc_ref[:SCP, :] = lax.dot_general(psc_ref[...], x_ref[...], dn,
                                        preferred_element_type=jnp.float32)
    btsc_ref[SCP:, :] = lax.dot_general(psc_ref[...], y_ref[...], dn,
                                        preferred_element_type=jnp.float32)
    btc_ref[:, :TCP] = jnp.dot(x_ref[...], ptc_ref[...],
                               preferred_element_type=jnp.float32)
    btc_ref[:, TCP:] = jnp.dot(y_ref[...], ptc_ref[...],
                               preferred_element_type=jnp.float32)


def _project(X, Y, proj):
    vmem = pl.BlockSpec(memory_space=pltpu.VMEM)
    return pl.pallas_call(
        _mm_body,
        in_specs=[vmem, vmem, vmem, vmem],
        out_specs=[pl.BlockSpec(memory_space=pltpu.VMEM)] * 2,
        out_shape=[jax.ShapeDtypeStruct((2 * SCP, N), jnp.float32),
                   jax.ShapeDtypeStruct((N, CT), jnp.float32)],
    )(X, Y, proj[:, :SCP], proj[:, SCP:])


def _full(v):
    return jnp.full((16,), v, jnp.int32)


def _transform(fbuf, dst):
    def body(i, carry):
        sl = pl.ds(i * 16, 16)
        u = lax.bitcast_convert_type(fbuf[sl], jnp.int32)
        s = lax.shift_right_arithmetic(u, _full(31))
        dst[sl] = lax.bitwise_xor(u, lax.bitwise_or(s, _full(MININT)))
        return carry
    lax.fori_loop(0, NV, body, 0, unroll=8)


def _hist_pass(src, hist, sh, base, iota16):
    def body(i, carry):
        k = plsc.load_gather(src, [base + i])
        d = lax.bitwise_and(lax.shift_right_logical(k, _full(sh)), _full(255))
        plsc.addupdate_scatter(hist, [d * 16 + iota16], _full(1))
        return carry
    lax.fori_loop(0, NV, body, 0, unroll=8)


def _offsets(hist, off):
    def body(v, carry):
        sl = pl.ds(v * 16, 16)
        h = hist[sl]
        cs = plsc.cumsum(h)
        off[sl] = cs - h + carry
        hist[sl] = _full(0)
        return carry + jnp.sum(h)
    lax.fori_loop(0, NV, body, jnp.int32(0), unroll=8)


def _permute2(sx, dx, ox, sy, dy, oy, sh, base, iota16):
    def one(src, dst, off, i):
        k = plsc.load_gather(src, [base + i])
        d = lax.bitwise_and(lax.shift_right_logical(k, _full(sh)), _full(255))
        didx = d * 16 + iota16
        pos = lax.bitwise_and(plsc.load_gather(off, [didx]), _full(N - 1))
        plsc.store_scatter(dst, [pos], k)
        plsc.addupdate_scatter(off, [didx], _full(1))

    def body(i, carry):
        one(sx, dx, ox, i)
        one(sy, dy, oy, i)
        return carry
    lax.fori_loop(0, NV, body, 0, unroll=8)


def _sort_pair(fxq, fyq, a, b, c, d_, hx, hy, ox, oy, base, iota16):
    _transform(fxq, a)
    _transform(fyq, c)
    _hist_pass(a, hx, 0, base, iota16)
    _hist_pass(c, hy, 0, base, iota16)
    _offsets(hx, ox)
    _offsets(hy, oy)
    _permute2(a, b, ox, c, d_, oy, 0, base, iota16)
    for sh, sx, dx, sy, dy in ((8, b, a, d_, c), (16, a, b, c, d_),
                               (24, b, a, d_, c)):
        _hist_pass(sx, hx, sh, base, iota16)
        _hist_pass(sy, hy, sh, base, iota16)
        _offsets(hx, ox)
        _offsets(hy, oy)
        _permute2(sx, dx, ox, sy, dy, oy, sh, base, iota16)


def _k2f(k):
    s = lax.shift_right_arithmetic(lax.bitwise_not(k), _full(31))
    return lax.bitcast_convert_type(
        lax.bitwise_xor(k, lax.bitwise_or(s, _full(MININT))), jnp.float32)


def _cost(a, c):
    def body(i, acc):
        sl = pl.ds(i * 16, 16)
        d = _k2f(a[sl]) - _k2f(c[sl])
        return acc + d * d
    return lax.fori_loop(0, NV, body, jnp.zeros((16,), jnp.float32), unroll=8)


def _sc_body(bt_hbm, out_hbm, fx, fy, a, b, c, d_, hx, hy, ox, oy, accv):
    wid = lax.axis_index("s") * 2 + lax.axis_index("c")
    iota16 = lax.iota(jnp.int32, 16)
    base = iota16 * NV

    pltpu.sync_copy(bt_hbm.at[pl.ds(wid * N, N)], fx)
    pltpu.sync_copy(bt_hbm.at[pl.ds((SCP + wid) * N, N)], fy)

    def clr(i, carry):
        hx[pl.ds(i * 16, 16)] = _full(0)
        hy[pl.ds(i * 16, 16)] = _full(0)
        return carry
    lax.fori_loop(0, NV, clr, 0, unroll=8)

    _sort_pair(fx, fy, a, b, c, d_, hx, hy, ox, oy, base, iota16)
    accv[...] = _cost(a, c)

    pltpu.sync_copy(accv, out_hbm.at[pl.ds(wid * 16, 16)])


def _sc_sort_cost(BT):
    mesh = plsc.VectorSubcoreMesh(core_axis_name="c", subcore_axis_name="s")
    f = functools.partial(
        pl.kernel,
        mesh=mesh,
        compiler_params=pltpu.CompilerParams(needs_layout_passes=False),
        out_type=jax.ShapeDtypeStruct((NW * 16,), jnp.float32),
        scratch_types=[
            pltpu.VMEM((N,), jnp.float32),
            pltpu.VMEM((N,), jnp.float32),
            pltpu.VMEM((N,), jnp.int32),
            pltpu.VMEM((N,), jnp.int32),
            pltpu.VMEM((N,), jnp.int32),
            pltpu.VMEM((N,), jnp.int32),
            pltpu.VMEM((N,), jnp.int32),
            pltpu.VMEM((N,), jnp.int32),
            pltpu.VMEM((N,), jnp.int32),
            pltpu.VMEM((N,), jnp.int32),
            pltpu.VMEM((16,), jnp.float32),
        ],
    )(_sc_body)
    return f(BT.reshape(2 * SCP * N))


def _substage(b_ref, kbit, d):
    m = N // (2 * d)
    rows = lax.broadcasted_iota(jnp.int32, (N, 1), 0)
    asc = ((rows >> kbit) & 1) == 0
    v = b_ref[...].reshape(m, 2 * d, CT)
    a3 = asc.reshape(m, 2 * d, 1)
    asc_b = a3[:, 0:1, :]
    lo = v[:, :d, :]
    hi = v[:, d:, :]
    mn = jnp.minimum(lo, hi)
    mx = jnp.maximum(lo, hi)
    new_lo = jnp.where(asc_b, mn, mx)
    new_hi = jnp.where(asc_b, mx, mn)
    b_ref[...] = jnp.concatenate([new_lo, new_hi], axis=1).reshape(N, CT)


def _bitonic_body(jj_ref, kk_ref, in_ref, out_ref, b_ref):
    s = pl.program_id(0)

    @pl.when(s == 0)
    def _init():
        b_ref[...] = in_ref[...]

    @pl.when(s > 0)
    def _sort():
        t = s - 1
        jv = jj_ref[t]
        kv = kk_ref[t]
        branches = [functools.partial(_substage, b_ref, kv, 1 << j)
                    for j in range(12)]
        lax.switch(jv, branches)

    @pl.when(s == N_SUB)
    def _reduce():
        diff = b_ref[:, :TCP] - b_ref[:, TCP:]
        out_ref[...] = jnp.sum(diff * diff, keepdims=True).reshape(1, 1)


def _bitonic_cost(Btc):
    jj = jnp.array([j for _, j in _SCHED], dtype=jnp.int32)
    kk = jnp.array([k for k, _ in _SCHED], dtype=jnp.int32)
    smem = pl.BlockSpec(memory_space=pltpu.SMEM)
    vmem = pl.BlockSpec(memory_space=pltpu.VMEM)
    return pl.pallas_call(
        _bitonic_body,
        grid=(N_SUB + 1,),
        in_specs=[smem, smem, vmem],
        out_specs=pl.BlockSpec(memory_space=pltpu.VMEM),
        out_shape=jax.ShapeDtypeStruct((1, 1), jnp.float32),
        scratch_shapes=[pltpu.VMEM((N, CT), jnp.float32)],
    )(jj, kk, Btc)


def _fin_body(t_ref, s_ref, o_ref):
    tot = t_ref[0, 0] + jnp.sum(s_ref[...])
    o_ref[...] = jnp.sqrt((tot * (1.0 / (N * P))) *
                          jnp.ones((1, 1), jnp.float32))


def _finalize(tc_cost, sc_costs):
    vmem = pl.BlockSpec(memory_space=pltpu.VMEM)
    return pl.pallas_call(
        _fin_body,
        in_specs=[vmem, vmem],
        out_specs=pl.BlockSpec(memory_space=pltpu.VMEM),
        out_shape=jax.ShapeDtypeStruct((1, 1), jnp.float32),
    )(tc_cost, sc_costs.reshape(8, 64))


def kernel(X, Y):
    proj = _projections()
    BTsc, Btc = _project(X, Y, proj)
    sc_costs = _sc_sort_cost(BTsc)
    tc_cost = _bitonic_cost(Btc)
    return _finalize(tc_cost, sc_costs)[0, 0]
